# Initial kernel scaffold; baseline (speedup 1.0000x reference)
import jax
import jax.numpy as jnp
from jax.experimental import pallas as pl


def kernel(H, noise, unif, Wm, bm, Ws, bs, Wq, bq):
    raise NotImplementedError("write your pallas kernel here")


# speedup vs baseline: 1.5745x; 1.5745x over previous
import functools

import jax
import jax.numpy as jnp
from jax.experimental import pallas as pl
from jax.experimental.pallas import tpu as pltpu


def _vi_kernel(h_ref, noise_ref, unif_ref, w_ref, b_ref,
               mean_ref, logstd_ref, q_ref, m_ref,
               *, inv_temp, out_dim, cat, dd):
    h = h_ref[...].astype(jnp.bfloat16)
    fused = jnp.dot(h, w_ref[...], preferred_element_type=jnp.float32) + b_ref[...]
    mean = fused[:, :out_dim]
    logstd = fused[:, out_dim:2 * out_dim]
    q = fused[:, 2 * out_dim:2 * out_dim + cat]

    n = noise_ref[...] * jnp.exp(logstd) + mean

    eps = 1e-07
    u = unif_ref[...]
    gumbel = -jnp.log(-jnp.log(u + eps) + eps)
    logits = (q + gumbel) * inv_temp
    logits = logits - jnp.max(logits, axis=-1, keepdims=True)
    ez = jnp.exp(logits)
    z = ez * pl.reciprocal(jnp.sum(ez, axis=-1, keepdims=True), approx=True)

    acc = jnp.zeros((h_ref.shape[0], dd), jnp.float32)
    for c in range(cat):
        acc = acc + z[:, c:c + 1] * n[:, c * dd:(c + 1) * dd]

    mean_ref[...] = mean
    logstd_ref[...] = logstd
    q_ref[...] = q
    m_ref[...] = acc


def _plan_rows(P, tm):
    if P >= 16:
        tm = min(tm, pl.cdiv(P, 2))
    tm = max(8, ((min(tm, P) + 7) // 8) * 8)
    grid = pl.cdiv(P, tm)
    return tm, grid, grid * tm


@functools.partial(jax.jit, static_argnames=("temp", "cat", "tm"))
def _vi_forward(H, noise, unif, Wm, bm, Ws, bs, Wq, bq, *, temp, cat, tm=256):
    P, in_dim = H.shape
    out_dim = Wm.shape[0]
    dd = out_dim // cat
    n_fused = 2 * out_dim + cat

    w_fused = jnp.concatenate([Wm.T, Ws.T, Wq.T], axis=1).astype(jnp.bfloat16)
    b_fused = jnp.concatenate([bm, bs, bq]).reshape(1, n_fused).astype(jnp.float32)

    tm, grid, P_pad = _plan_rows(P, tm)
    pad = P_pad - P
    if pad:
        H = jnp.pad(H, ((0, pad), (0, 0)))
        noise = jnp.pad(noise, ((0, pad), (0, 0)))
        unif = jnp.pad(unif, ((0, pad), (0, 0)), constant_values=0.5)

    _kernel_fn = functools.partial(_vi_kernel, inv_temp=float(1.0 / temp),
                                   out_dim=out_dim, cat=cat, dd=dd)
    mean, logstd, q, M = pl.pallas_call(
        _kernel_fn,
        out_shape=(
            jax.ShapeDtypeStruct((P_pad, out_dim), jnp.float32),
            jax.ShapeDtypeStruct((P_pad, out_dim), jnp.float32),
            jax.ShapeDtypeStruct((P_pad, cat), jnp.float32),
            jax.ShapeDtypeStruct((P_pad, dd), jnp.float32),
        ),
        grid=(grid,),
        in_specs=[
            pl.BlockSpec((tm, in_dim), lambda i: (i, 0)),
            pl.BlockSpec((tm, out_dim), lambda i: (i, 0)),
            pl.BlockSpec((tm, cat), lambda i: (i, 0)),
            pl.BlockSpec((in_dim, n_fused), lambda i: (0, 0)),
            pl.BlockSpec((1, n_fused), lambda i: (0, 0)),
        ],
        out_specs=(
            pl.BlockSpec((tm, out_dim), lambda i: (i, 0)),
            pl.BlockSpec((tm, out_dim), lambda i: (i, 0)),
            pl.BlockSpec((tm, cat), lambda i: (i, 0)),
            pl.BlockSpec((tm, dd), lambda i: (i, 0)),
        ),
        compiler_params=pltpu.CompilerParams(
            dimension_semantics=("parallel",),
            vmem_limit_bytes=64 * 1024 * 1024,
        ),
    )(H, noise, unif, w_fused, b_fused)

    if pad:
        mean, logstd, q, M = mean[:P], logstd[:P], q[:P], M[:P]
    return M, mean, logstd, q


def kernel(H, noise, unif, Wm, bm, Ws, bs, Wq, bq):
    return _vi_forward(H, noise, unif, Wm, bm, Ws, bs, Wq, bq, temp=0.5, cat=4)


# speedup vs baseline: 1.9992x; 1.2697x over previous
import functools

import jax
import jax.numpy as jnp
from jax.experimental import pallas as pl
from jax.experimental.pallas import tpu as pltpu


def _vi_kernel(h_ref, noise_ref, unif_ref, w_ref, b_ref,
               mean_ref, logstd_ref, q_ref, m_ref,
               *, inv_temp, out_dim, cat, dd):
    h = h_ref[...].astype(jnp.bfloat16)
    fused = jnp.dot(h, w_ref[...], preferred_element_type=jnp.float32) + b_ref[...]
    mean = fused[:, :out_dim]
    logstd = fused[:, out_dim:2 * out_dim]
    q = fused[:, 2 * out_dim:2 * out_dim + cat]

    n = noise_ref[...] * jnp.exp(logstd) + mean

    eps = 1e-07
    u = unif_ref[...]
    gumbel = -jnp.log(-jnp.log(u + eps) + eps)
    logits = (q + gumbel) * inv_temp
    logits = logits - jnp.max(logits, axis=-1, keepdims=True)
    ez = jnp.exp(logits)
    z = ez * pl.reciprocal(jnp.sum(ez, axis=-1, keepdims=True), approx=True)

    acc = jnp.zeros((h_ref.shape[0], dd), jnp.float32)
    for c in range(cat):
        acc = acc + z[:, c:c + 1] * n[:, c * dd:(c + 1) * dd]

    mean_ref[...] = mean
    logstd_ref[...] = logstd
    q_ref[...] = q
    m_ref[...] = acc


def _plan_rows(P, tm):
    if P >= 16:
        tm = min(tm, pl.cdiv(P, 2))
    tm = max(8, ((min(tm, P) + 7) // 8) * 8)
    grid = pl.cdiv(P, tm)
    return tm, grid, grid * tm


@functools.partial(jax.jit, static_argnames=("temp", "cat", "tm"))
def _vi_forward(H, noise, unif, Wm, bm, Ws, bs, Wq, bq, *, temp, cat, tm=256):
    P, in_dim = H.shape
    out_dim = Wm.shape[0]
    dd = out_dim // cat
    n_fused = 2 * out_dim + cat

    w_fused = jnp.concatenate([Wm.T, Ws.T, Wq.T], axis=1).astype(jnp.bfloat16)
    b_fused = jnp.concatenate([bm, bs, bq]).reshape(1, n_fused).astype(jnp.float32)

    tm, grid, P_pad = _plan_rows(P, tm)
    pad = P_pad - P
    if pad:
        H = jnp.pad(H, ((0, pad), (0, 0)))
        noise = jnp.pad(noise, ((0, pad), (0, 0)))
        unif = jnp.pad(unif, ((0, pad), (0, 0)), constant_values=0.5)

    _kernel_fn = functools.partial(_vi_kernel, inv_temp=float(1.0 / temp),
                                   out_dim=out_dim, cat=cat, dd=dd)
    mean, logstd, q, M = pl.pallas_call(
        _kernel_fn,
        out_shape=(
            jax.ShapeDtypeStruct((P_pad, out_dim), jnp.float32),
            jax.ShapeDtypeStruct((P_pad, out_dim), jnp.float32),
            jax.ShapeDtypeStruct((P_pad, cat), jnp.float32),
            jax.ShapeDtypeStruct((P_pad, dd), jnp.float32),
        ),
        grid=(grid,),
        in_specs=[
            pl.BlockSpec((tm, in_dim), lambda i: (i, 0)),
            pl.BlockSpec((tm, out_dim), lambda i: (i, 0)),
            pl.BlockSpec((tm, cat), lambda i: (i, 0)),
            pl.BlockSpec((in_dim, n_fused), lambda i: (0, 0)),
            pl.BlockSpec((1, n_fused), lambda i: (0, 0)),
        ],
        out_specs=(
            pl.BlockSpec((tm, out_dim), lambda i: (i, 0)),
            pl.BlockSpec((tm, out_dim), lambda i: (i, 0)),
            pl.BlockSpec((tm, cat), lambda i: (i, 0)),
            pl.BlockSpec((tm, dd), lambda i: (i, 0)),
        ),
        compiler_params=pltpu.CompilerParams(
            dimension_semantics=("parallel",),
            vmem_limit_bytes=64 * 1024 * 1024,
        ),
    )(H, noise, unif, w_fused, b_fused)

    if pad:
        mean, logstd, q, M = mean[:P], logstd[:P], q[:P], M[:P]
    return M, mean, logstd, q


def kernel(H, noise, unif, Wm, bm, Ws, bs, Wq, bq):
    return _vi_forward(H, noise, unif, Wm, bm, Ws, bs, Wq, bq, temp=0.5, cat=4,
                       tm=512)
